# Initial kernel scaffold; baseline (speedup 1.0000x reference)
import functools

import jax
import jax.numpy as jnp
from jax import lax
from jax.experimental import pallas as pl
from jax.experimental.pallas import tpu as pltpu
from jax.experimental.pallas import tpu_sc as plsc

_NC = 2
_NS = 16
_LANES = 16


def _edge_embed(edge_attr, wt, bias):
    E, D = edge_attr.shape
    L = wt.shape[0]
    TE = 512
    grid = E // TE

    def ek(x_ref, w_ref, b_ref, *outs):
        x = x_ref[...]
        for l in range(L):
            outs[l][...] = (
                jnp.dot(x, w_ref[l], preferred_element_type=jnp.float32)
                + b_ref[l]
            )

    return pl.pallas_call(
        ek,
        grid=(grid,),
        in_specs=[
            pl.BlockSpec((TE, D), lambda i: (i, 0)),
            pl.BlockSpec((L, D, D), lambda i: (0, 0, 0)),
            pl.BlockSpec((L, 1, D), lambda i: (0, 0, 0)),
        ],
        out_specs=[pl.BlockSpec((TE, D), lambda i: (i, 0))] * L,
        out_shape=[jax.ShapeDtypeStruct((E, D), jnp.float32)] * L,
    )(edge_attr, wt, bias)


def _make_sc_edge(N, D, E):
    NW = _NC * _NS
    per_w = E // NW
    C = 80
    n_chunks = per_w // C
    rows_per_s = N // _NS

    mesh = plsc.VectorSubcoreMesh(core_axis_name="c", subcore_axis_name="s")

    @functools.partial(
        pl.kernel,
        mesh=mesh,
        out_type=jax.ShapeDtypeStruct((_NC, N, D), jnp.float32),
        scratch_types=[
            pltpu.VMEM((C,), jnp.int32),
            pltpu.VMEM((C,), jnp.int32),
            pltpu.VMEM((C, D), jnp.float32),
            pltpu.VMEM((C, D), jnp.float32),
            pltpu.VMEM_SHARED((N, D), jnp.float32),
            pltpu.SemaphoreType.DMA,
        ],
    )
    def sc_edge(h_hbm, e_hbm, src_hbm, dst_hbm, zeros_hbm, out_hbm,
                src_v, dst_v, hrow_v, e_v, agg_sh, sem):
        cid = lax.axis_index("c")
        sid = lax.axis_index("s")
        wid = sid * _NC + cid
        row0 = sid * rows_per_s

        pltpu.sync_copy(zeros_hbm.at[pl.ds(row0, rows_per_s)],
                        agg_sh.at[pl.ds(row0, rows_per_s)])
        plsc.subcore_barrier()

        def chunk(k, carry):
            base = wid * per_w + k * C
            pltpu.sync_copy(src_hbm.at[pl.ds(base, C)], src_v)
            pltpu.sync_copy(dst_hbm.at[pl.ds(base, C)], dst_v)
            gather = pltpu.async_copy(h_hbm.at[src_v], hrow_v, sem)
            pltpu.sync_copy(e_hbm.at[pl.ds(base, C)], e_v)
            gather.wait()

            def row(i, cc):
                for j in range(D // _LANES):
                    sl = pl.ds(j * _LANES, _LANES)
                    e_v[i, sl] = jnp.maximum(hrow_v[i, sl] + e_v[i, sl], 0.0)
                return cc

            lax.fori_loop(0, C, row, 0)
            pltpu.sync_copy(e_v, agg_sh.at[dst_v], add=True)
            return carry

        lax.fori_loop(0, n_chunks, chunk, 0)
        plsc.subcore_barrier()
        pltpu.sync_copy(agg_sh.at[pl.ds(row0, rows_per_s)],
                        out_hbm.at[cid, pl.ds(row0, rows_per_s)])

    return sc_edge


def _node_mlp(h, agg, w1t, b1, g1, bb1, w2t, b2, g2, bb2, final_relu):
    Nn, D = h.shape

    def nk(h_ref, a_ref, w1_ref, b1_ref, g1_ref, bb1_ref,
           w2_ref, b2_ref, g2_ref, bb2_ref, o_ref):
        z = h_ref[...] + a_ref[0] + a_ref[1]
        z = jnp.dot(z, w1_ref[...], preferred_element_type=jnp.float32) + b1_ref[...]
        m = jnp.mean(z, axis=0, keepdims=True)
        v = jnp.mean(jnp.square(z - m), axis=0, keepdims=True)
        z = (z - m) * lax.rsqrt(v + 1e-5) * g1_ref[...] + bb1_ref[...]
        z = jnp.maximum(z, 0.0)
        z = jnp.dot(z, w2_ref[...], preferred_element_type=jnp.float32) + b2_ref[...]
        m = jnp.mean(z, axis=0, keepdims=True)
        v = jnp.mean(jnp.square(z - m), axis=0, keepdims=True)
        z = (z - m) * lax.rsqrt(v + 1e-5) * g2_ref[...] + bb2_ref[...]
        if final_relu:
            z = jnp.maximum(z, 0.0)
        o_ref[...] = z

    return pl.pallas_call(
        nk,
        out_shape=jax.ShapeDtypeStruct((Nn, D), jnp.float32),
    )(h, agg, w1t, b1, g1, bb1, w2t, b2, g2, bb2)


def kernel(x, edge_index, edge_attr, batch,
           lin_edge_w, lin_edge_b, mlp_w1, mlp_b1, mlp_bn_g, mlp_bn_b,
           mlp_w2, mlp_b2, norm_g, norm_b):
    N, D = x.shape
    E = edge_index.shape[1]
    L = lin_edge_w.shape[0]

    src = edge_index[0]
    dst = edge_index[1]

    wt_e = jnp.transpose(lin_edge_w, (0, 2, 1))
    be = lin_edge_b[:, None, :]
    e_list = _edge_embed(edge_attr, wt_e, be)

    zeros = jnp.zeros((N, D), jnp.float32)
    sc_edge = _make_sc_edge(N, D, E)

    h = x
    for l in range(L):
        agg = sc_edge(h, e_list[l], src, dst, zeros)
        h = _node_mlp(
            h, agg,
            mlp_w1[l].T, mlp_b1[l][None], mlp_bn_g[l][None], mlp_bn_b[l][None],
            mlp_w2[l].T, mlp_b2[l][None], norm_g[l][None], norm_b[l][None],
            l < L - 1,
        )
    return h


# speedup vs baseline: 2.6430x; 2.6430x over previous
import functools

import jax
import jax.numpy as jnp
from jax import lax
from jax.experimental import pallas as pl
from jax.experimental.pallas import tpu as pltpu
from jax.experimental.pallas import tpu_sc as plsc

_NC = 2
_NS = 16
_LANES = 16


def _edge_embed(edge_attr, wt, bias):
    E, D = edge_attr.shape
    L = wt.shape[0]
    TE = 512
    grid = E // TE

    def ek(x_ref, w_ref, b_ref, *outs):
        x = x_ref[...]
        for l in range(L):
            outs[l][...] = (
                jnp.dot(x, w_ref[l], preferred_element_type=jnp.float32)
                + b_ref[l]
            )

    return pl.pallas_call(
        ek,
        grid=(grid,),
        in_specs=[
            pl.BlockSpec((TE, D), lambda i: (i, 0)),
            pl.BlockSpec((L, D, D), lambda i: (0, 0, 0)),
            pl.BlockSpec((L, 1, D), lambda i: (0, 0, 0)),
        ],
        out_specs=[pl.BlockSpec((TE, D), lambda i: (i, 0))] * L,
        out_shape=[jax.ShapeDtypeStruct((E, D), jnp.float32)] * L,
    )(edge_attr, wt, bias)


def _make_sc_edge(N, D, E):
    NW = _NC * _NS
    per_w = E // NW
    C = 80
    n_chunks = per_w // C
    rows_per_s = (N // _NS) // 8 * 8
    tail_rows = N - rows_per_s * _NS

    mesh = plsc.VectorSubcoreMesh(core_axis_name="c", subcore_axis_name="s")

    @functools.partial(
        pl.kernel,
        mesh=mesh,
        out_type=jax.ShapeDtypeStruct((_NC, N, D), jnp.float32),
        scratch_types=[
            pltpu.VMEM((C,), jnp.int32),
            pltpu.VMEM((C,), jnp.int32),
            pltpu.VMEM((C, D), jnp.float32),
            pltpu.VMEM((C, D), jnp.float32),
            pltpu.VMEM_SHARED((N, D), jnp.float32),
            pltpu.SemaphoreType.DMA,
        ],
    )
    def sc_edge(h_hbm, e_hbm, src_hbm, dst_hbm, zeros_hbm, out_hbm,
                src_v, dst_v, hrow_v, e_v, agg_sh, sem):
        cid = lax.axis_index("c")
        sid = lax.axis_index("s")
        wid = sid * _NC + cid
        row0 = sid * rows_per_s

        pltpu.sync_copy(zeros_hbm.at[pl.ds(row0, rows_per_s)],
                        agg_sh.at[pl.ds(row0, rows_per_s)])
        if tail_rows:
            @pl.when(sid == _NS - 1)
            def _zero_tail():
                pltpu.sync_copy(zeros_hbm.at[pl.ds(rows_per_s * _NS, tail_rows)],
                                agg_sh.at[pl.ds(rows_per_s * _NS, tail_rows)])
        plsc.subcore_barrier()

        def chunk(k, carry):
            base = wid * per_w + k * C
            pltpu.sync_copy(src_hbm.at[pl.ds(base, C)], src_v)
            pltpu.sync_copy(dst_hbm.at[pl.ds(base, C)], dst_v)
            gather = pltpu.async_copy(h_hbm.at[src_v], hrow_v, sem)
            pltpu.sync_copy(e_hbm.at[pl.ds(base, C)], e_v)
            gather.wait()

            def row(i, cc):
                for j in range(D // _LANES):
                    sl = pl.ds(j * _LANES, _LANES)
                    e_v[i, sl] = jnp.maximum(hrow_v[i, sl] + e_v[i, sl], 0.0)
                return cc

            lax.fori_loop(0, C, row, 0)
            pltpu.sync_copy(e_v, agg_sh.at[dst_v], add=True)
            return carry

        lax.fori_loop(0, n_chunks, chunk, 0)
        plsc.subcore_barrier()
        pltpu.sync_copy(agg_sh.at[pl.ds(row0, rows_per_s)],
                        out_hbm.at[cid, pl.ds(row0, rows_per_s)])
        if tail_rows:
            @pl.when(sid == _NS - 1)
            def _drain_tail():
                pltpu.sync_copy(agg_sh.at[pl.ds(rows_per_s * _NS, tail_rows)],
                                out_hbm.at[cid, pl.ds(rows_per_s * _NS, tail_rows)])

    return sc_edge


def _node_mlp(h, agg, w1t, b1, g1, bb1, w2t, b2, g2, bb2, final_relu):
    Nn, D = h.shape

    def nk(h_ref, a_ref, w1_ref, b1_ref, g1_ref, bb1_ref,
           w2_ref, b2_ref, g2_ref, bb2_ref, o_ref):
        z = h_ref[...] + a_ref[0] + a_ref[1]
        z = jnp.dot(z, w1_ref[...], preferred_element_type=jnp.float32) + b1_ref[...]
        m = jnp.mean(z, axis=0, keepdims=True)
        v = jnp.mean(jnp.square(z - m), axis=0, keepdims=True)
        z = (z - m) * lax.rsqrt(v + 1e-5) * g1_ref[...] + bb1_ref[...]
        z = jnp.maximum(z, 0.0)
        z = jnp.dot(z, w2_ref[...], preferred_element_type=jnp.float32) + b2_ref[...]
        m = jnp.mean(z, axis=0, keepdims=True)
        v = jnp.mean(jnp.square(z - m), axis=0, keepdims=True)
        z = (z - m) * lax.rsqrt(v + 1e-5) * g2_ref[...] + bb2_ref[...]
        if final_relu:
            z = jnp.maximum(z, 0.0)
        o_ref[...] = z

    return pl.pallas_call(
        nk,
        out_shape=jax.ShapeDtypeStruct((Nn, D), jnp.float32),
    )(h, agg, w1t, b1, g1, bb1, w2t, b2, g2, bb2)


def kernel(x, edge_index, edge_attr, batch,
           lin_edge_w, lin_edge_b, mlp_w1, mlp_b1, mlp_bn_g, mlp_bn_b,
           mlp_w2, mlp_b2, norm_g, norm_b):
    N, D = x.shape
    E = edge_index.shape[1]
    L = lin_edge_w.shape[0]

    src = edge_index[0]
    dst = edge_index[1]

    wt_e = jnp.transpose(lin_edge_w, (0, 2, 1))
    be = lin_edge_b[:, None, :]
    e_list = _edge_embed(edge_attr, wt_e, be)

    zeros = jnp.zeros((N, D), jnp.float32)
    sc_edge = _make_sc_edge(N, D, E)

    h = x
    for l in range(L):
        agg = sc_edge(h, e_list[l], src, dst, zeros)
        h = _node_mlp(
            h, agg,
            mlp_w1[l].T, mlp_b1[l][None], mlp_bn_g[l][None], mlp_bn_b[l][None],
            mlp_w2[l].T, mlp_b2[l][None], norm_g[l][None], norm_b[l][None],
            l < L - 1,
        )
    return h


# speedup vs baseline: 3.5601x; 1.3470x over previous
import functools

import jax
import jax.numpy as jnp
from jax import lax
from jax.experimental import pallas as pl
from jax.experimental.pallas import tpu as pltpu
from jax.experimental.pallas import tpu_sc as plsc

_NC = 2
_NS = 16
_LANES = 16
_CHUNK = 80


def _edge_embed(edge_attr, wt, bias):
    E, D = edge_attr.shape
    L = wt.shape[0]
    TE = 512
    grid = E // TE

    def ek(x_ref, w_ref, b_ref, *outs):
        x = x_ref[...]
        for l in range(L):
            outs[l][...] = (
                jnp.dot(x, w_ref[l], preferred_element_type=jnp.float32)
                + b_ref[l]
            )

    return pl.pallas_call(
        ek,
        grid=(grid,),
        in_specs=[
            pl.BlockSpec((TE, D), lambda i: (i, 0)),
            pl.BlockSpec((L, D, D), lambda i: (0, 0, 0)),
            pl.BlockSpec((L, 1, D), lambda i: (0, 0, 0)),
        ],
        out_specs=[pl.BlockSpec((TE, D), lambda i: (i, 0))] * L,
        out_shape=[jax.ShapeDtypeStruct((E, D), jnp.float32)] * L,
    )(edge_attr, wt, bias)


def _make_sc_edge(N, D, E):
    NW = _NC * _NS
    per_w = E // NW
    C = _CHUNK
    n_chunks = per_w // C
    n_pairs = n_chunks // 2
    rows_per_s = (N // _NS) // 8 * 8
    tail_rows = N - rows_per_s * _NS

    mesh = plsc.VectorSubcoreMesh(core_axis_name="c", subcore_axis_name="s")

    @functools.partial(
        pl.kernel,
        mesh=mesh,
        out_type=jax.ShapeDtypeStruct((_NC, N, D), jnp.float32),
        scratch_types=[
            pltpu.VMEM((2, C), jnp.int32),
            pltpu.VMEM((2, C), jnp.int32),
            pltpu.VMEM((2, C, D), jnp.float32),
            pltpu.VMEM((2, C, D), jnp.float32),
            pltpu.VMEM_SHARED((N, D), jnp.float32),
            pltpu.SemaphoreType.DMA,
            pltpu.SemaphoreType.DMA,
            pltpu.SemaphoreType.DMA,
            pltpu.SemaphoreType.DMA,
        ],
    )
    def sc_edge(h_hbm, e_hbm, src_hbm, dst_hbm, zeros_hbm, out_hbm,
                src_v, dst_v, hrow_v, e_v, agg_sh, gs0, gs1, es0, es1):
        cid = lax.axis_index("c")
        sid = lax.axis_index("s")
        wid = sid * _NC + cid
        row0 = sid * rows_per_s
        gsems = (gs0, gs1)
        esems = (es0, es1)

        pltpu.sync_copy(zeros_hbm.at[pl.ds(row0, rows_per_s)],
                        agg_sh.at[pl.ds(row0, rows_per_s)])
        if tail_rows:
            @pl.when(sid == _NS - 1)
            def _zero_tail():
                pltpu.sync_copy(zeros_hbm.at[pl.ds(rows_per_s * _NS, tail_rows)],
                                agg_sh.at[pl.ds(rows_per_s * _NS, tail_rows)])
        plsc.subcore_barrier()

        def load_idx(k, b):
            base = wid * per_w + k * C
            pltpu.sync_copy(src_hbm.at[pl.ds(base, C)], src_v.at[b])
            pltpu.sync_copy(dst_hbm.at[pl.ds(base, C)], dst_v.at[b])

        def start(k, b):
            pltpu.async_copy(h_hbm.at[src_v.at[b]], hrow_v.at[b], gsems[b])
            pltpu.async_copy(e_hbm.at[pl.ds(wid * per_w + k * C, C)],
                             e_v.at[b], esems[b])

        def finish(k, b):
            pltpu.make_async_copy(h_hbm.at[src_v.at[b]], hrow_v.at[b],
                                  gsems[b]).wait()
            pltpu.make_async_copy(e_hbm.at[pl.ds(wid * per_w + k * C, C)],
                                  e_v.at[b], esems[b]).wait()

        def compute(k, b):
            @plsc.parallel_loop(0, C, unroll=4)
            def _row(i):
                for j in range(D // _LANES):
                    sl = pl.ds(j * _LANES, _LANES)
                    e_v[b, i, sl] = jnp.maximum(hrow_v[b, i, sl] + e_v[b, i, sl],
                                                0.0)
            pltpu.sync_copy(e_v.at[b], agg_sh.at[dst_v.at[b]], add=True)

        load_idx(0, 0)
        start(0, 0)

        def pair(i, carry):
            for b in range(2):
                k = 2 * i + b

                @pl.when(k + 1 < n_chunks)
                def _prefetch():
                    load_idx(k + 1, 1 - b)
                    start(k + 1, 1 - b)

                finish(k, b)
                compute(k, b)
            return carry

        lax.fori_loop(0, n_chunks // 2, pair, 0)
        if n_chunks % 2:
            finish(n_chunks - 1, 0)
            compute(n_chunks - 1, 0)

        plsc.subcore_barrier()
        pltpu.sync_copy(agg_sh.at[pl.ds(row0, rows_per_s)],
                        out_hbm.at[cid, pl.ds(row0, rows_per_s)])
        if tail_rows:
            @pl.when(sid == _NS - 1)
            def _drain_tail():
                pltpu.sync_copy(agg_sh.at[pl.ds(rows_per_s * _NS, tail_rows)],
                                out_hbm.at[cid, pl.ds(rows_per_s * _NS, tail_rows)])

    return sc_edge


def _node_mlp(h, agg, w1t, b1, g1, bb1, w2t, b2, g2, bb2, final_relu):
    Nn, D = h.shape

    def nk(h_ref, a_ref, w1_ref, b1_ref, g1_ref, bb1_ref,
           w2_ref, b2_ref, g2_ref, bb2_ref, o_ref):
        z = h_ref[...] + a_ref[0] + a_ref[1]
        z = jnp.dot(z, w1_ref[...], preferred_element_type=jnp.float32) + b1_ref[...]
        m = jnp.mean(z, axis=0, keepdims=True)
        v = jnp.mean(jnp.square(z - m), axis=0, keepdims=True)
        z = (z - m) * lax.rsqrt(v + 1e-5) * g1_ref[...] + bb1_ref[...]
        z = jnp.maximum(z, 0.0)
        z = jnp.dot(z, w2_ref[...], preferred_element_type=jnp.float32) + b2_ref[...]
        m = jnp.mean(z, axis=0, keepdims=True)
        v = jnp.mean(jnp.square(z - m), axis=0, keepdims=True)
        z = (z - m) * lax.rsqrt(v + 1e-5) * g2_ref[...] + bb2_ref[...]
        if final_relu:
            z = jnp.maximum(z, 0.0)
        o_ref[...] = z

    return pl.pallas_call(
        nk,
        out_shape=jax.ShapeDtypeStruct((Nn, D), jnp.float32),
    )(h, agg, w1t, b1, g1, bb1, w2t, b2, g2, bb2)


def kernel(x, edge_index, edge_attr, batch,
           lin_edge_w, lin_edge_b, mlp_w1, mlp_b1, mlp_bn_g, mlp_bn_b,
           mlp_w2, mlp_b2, norm_g, norm_b):
    N, D = x.shape
    E = edge_index.shape[1]
    L = lin_edge_w.shape[0]

    src = edge_index[0]
    dst = edge_index[1]

    wt_e = jnp.transpose(lin_edge_w, (0, 2, 1))
    be = lin_edge_b[:, None, :]
    e_list = _edge_embed(edge_attr, wt_e, be)

    zeros = jnp.zeros((N, D), jnp.float32)
    sc_edge = _make_sc_edge(N, D, E)

    h = x
    for l in range(L):
        agg = sc_edge(h, e_list[l], src, dst, zeros)
        h = _node_mlp(
            h, agg,
            mlp_w1[l].T, mlp_b1[l][None], mlp_bn_g[l][None], mlp_bn_b[l][None],
            mlp_w2[l].T, mlp_b2[l][None], norm_g[l][None], norm_b[l][None],
            l < L - 1,
        )
    return h


# speedup vs baseline: 4.1776x; 1.1735x over previous
import functools

import jax
import jax.numpy as jnp
from jax import lax
from jax.experimental import pallas as pl
from jax.experimental.pallas import tpu as pltpu
from jax.experimental.pallas import tpu_sc as plsc

_NC = 2
_NS = 16
_LANES = 16
_CHUNK = 40
_NBUF = 4


def _edge_embed(edge_attr, wt, bias):
    E, D = edge_attr.shape
    L = wt.shape[0]
    TE = 512
    grid = E // TE

    def ek(x_ref, w_ref, b_ref, *outs):
        x = x_ref[...]
        for l in range(L):
            outs[l][...] = (
                jnp.dot(x, w_ref[l], preferred_element_type=jnp.float32)
                + b_ref[l]
            )

    return pl.pallas_call(
        ek,
        grid=(grid,),
        in_specs=[
            pl.BlockSpec((TE, D), lambda i: (i, 0)),
            pl.BlockSpec((L, D, D), lambda i: (0, 0, 0)),
            pl.BlockSpec((L, 1, D), lambda i: (0, 0, 0)),
        ],
        out_specs=[pl.BlockSpec((TE, D), lambda i: (i, 0))] * L,
        out_shape=[jax.ShapeDtypeStruct((E, D), jnp.float32)] * L,
    )(edge_attr, wt, bias)


def _make_sc_edge(N, D, E):
    NW = _NC * _NS
    per_w = E // NW
    C = _CHUNK
    n_chunks = per_w // C
    n_pairs = n_chunks // 2
    rows_per_s = (N // _NS) // 8 * 8
    tail_rows = N - rows_per_s * _NS

    mesh = plsc.VectorSubcoreMesh(core_axis_name="c", subcore_axis_name="s")

    @functools.partial(
        pl.kernel,
        mesh=mesh,
        out_type=jax.ShapeDtypeStruct((_NC, N, D), jnp.float32),
        scratch_types=[
            pltpu.VMEM((_NBUF, C), jnp.int32),
            pltpu.VMEM((_NBUF, C), jnp.int32),
            pltpu.VMEM((_NBUF, C, D), jnp.float32),
            pltpu.VMEM((_NBUF, C, D), jnp.float32),
            pltpu.VMEM_SHARED((N, D), jnp.float32),
            pltpu.SemaphoreType.DMA((_NBUF,)),
            pltpu.SemaphoreType.DMA((_NBUF,)),
            pltpu.SemaphoreType.DMA((_NBUF,)),
        ],
    )
    def sc_edge(h_hbm, e_hbm, src_hbm, dst_hbm, zeros_hbm, out_hbm,
                src_v, dst_v, hrow_v, e_v, agg_sh, isem, dsem, ssem):
        cid = lax.axis_index("c")
        sid = lax.axis_index("s")
        wid = sid * _NC + cid
        row0 = sid * rows_per_s

        pltpu.sync_copy(zeros_hbm.at[pl.ds(row0, rows_per_s)],
                        agg_sh.at[pl.ds(row0, rows_per_s)])
        if tail_rows:
            @pl.when(sid == _NS - 1)
            def _zero_tail():
                pltpu.sync_copy(zeros_hbm.at[pl.ds(rows_per_s * _NS, tail_rows)],
                                agg_sh.at[pl.ds(rows_per_s * _NS, tail_rows)])
        plsc.subcore_barrier()

        def start_idx(k, b):
            base = wid * per_w + k * C
            pltpu.async_copy(src_hbm.at[pl.ds(base, C)], src_v.at[b], isem.at[b])
            pltpu.async_copy(dst_hbm.at[pl.ds(base, C)], dst_v.at[b], isem.at[b])

        def wait_idx(k, b):
            base = wid * per_w + k * C
            pltpu.make_async_copy(src_hbm.at[pl.ds(base, C)], src_v.at[b],
                                  isem.at[b]).wait()
            pltpu.make_async_copy(dst_hbm.at[pl.ds(base, C)], dst_v.at[b],
                                  isem.at[b]).wait()

        def start_data(k, b):
            pltpu.async_copy(h_hbm.at[src_v.at[b]], hrow_v.at[b], dsem.at[b])
            pltpu.async_copy(e_hbm.at[pl.ds(wid * per_w + k * C, C)],
                             e_v.at[b], dsem.at[b])

        def wait_data(k, b):
            pltpu.make_async_copy(h_hbm.at[src_v.at[b]], hrow_v.at[b],
                                  dsem.at[b]).wait()
            pltpu.make_async_copy(e_hbm.at[pl.ds(wid * per_w + k * C, C)],
                                  e_v.at[b], dsem.at[b]).wait()

        def wait_scatter(b):
            pltpu.make_async_copy(e_v.at[b], agg_sh.at[dst_v.at[b]],
                                  ssem.at[b]).wait()

        def compute(k, b):
            @plsc.parallel_loop(0, C, unroll=4)
            def _row(i):
                for j in range(D // _LANES):
                    sl = pl.ds(j * _LANES, _LANES)
                    e_v[b, i, sl] = jnp.maximum(hrow_v[b, i, sl] + e_v[b, i, sl],
                                                0.0)
            pltpu.async_copy(e_v.at[b], agg_sh.at[dst_v.at[b]], ssem.at[b],
                             add=True)

        start_idx(0, 0)
        wait_idx(0, 0)
        start_data(0, 0)
        start_idx(1, 1)

        n_groups = (n_chunks + _NBUF - 1) // _NBUF

        def group(i, carry):
            for p in range(_NBUF):
                k = _NBUF * i + p

                @pl.when(k >= 2)
                def _ws():
                    wait_scatter((p + 2) % _NBUF)

                @pl.when(k + 1 < n_chunks)
                def _pf():
                    b1 = (p + 1) % _NBUF
                    wait_idx(k + 1, b1)
                    start_data(k + 1, b1)

                @pl.when(k + 2 < n_chunks)
                def _pi():
                    start_idx(k + 2, (p + 2) % _NBUF)

                @pl.when(k < n_chunks)
                def _do():
                    wait_data(k, p)
                    compute(k, p)
            return carry

        lax.fori_loop(0, n_groups, group, 0)
        for k in range(max(_NBUF * n_groups - 2, 0), n_chunks):
            wait_scatter(k % _NBUF)

        plsc.subcore_barrier()
        pltpu.sync_copy(agg_sh.at[pl.ds(row0, rows_per_s)],
                        out_hbm.at[cid, pl.ds(row0, rows_per_s)])
        if tail_rows:
            @pl.when(sid == _NS - 1)
            def _drain_tail():
                pltpu.sync_copy(agg_sh.at[pl.ds(rows_per_s * _NS, tail_rows)],
                                out_hbm.at[cid, pl.ds(rows_per_s * _NS, tail_rows)])

    return sc_edge


def _node_mlp(h, agg, w1t, b1, g1, bb1, w2t, b2, g2, bb2, final_relu):
    Nn, D = h.shape

    def nk(h_ref, a_ref, w1_ref, b1_ref, g1_ref, bb1_ref,
           w2_ref, b2_ref, g2_ref, bb2_ref, o_ref):
        z = h_ref[...] + a_ref[0] + a_ref[1]
        z = jnp.dot(z, w1_ref[...], preferred_element_type=jnp.float32) + b1_ref[...]
        m = jnp.mean(z, axis=0, keepdims=True)
        v = jnp.mean(jnp.square(z - m), axis=0, keepdims=True)
        z = (z - m) * lax.rsqrt(v + 1e-5) * g1_ref[...] + bb1_ref[...]
        z = jnp.maximum(z, 0.0)
        z = jnp.dot(z, w2_ref[...], preferred_element_type=jnp.float32) + b2_ref[...]
        m = jnp.mean(z, axis=0, keepdims=True)
        v = jnp.mean(jnp.square(z - m), axis=0, keepdims=True)
        z = (z - m) * lax.rsqrt(v + 1e-5) * g2_ref[...] + bb2_ref[...]
        if final_relu:
            z = jnp.maximum(z, 0.0)
        o_ref[...] = z

    return pl.pallas_call(
        nk,
        out_shape=jax.ShapeDtypeStruct((Nn, D), jnp.float32),
    )(h, agg, w1t, b1, g1, bb1, w2t, b2, g2, bb2)


def kernel(x, edge_index, edge_attr, batch,
           lin_edge_w, lin_edge_b, mlp_w1, mlp_b1, mlp_bn_g, mlp_bn_b,
           mlp_w2, mlp_b2, norm_g, norm_b):
    N, D = x.shape
    E = edge_index.shape[1]
    L = lin_edge_w.shape[0]

    src = edge_index[0]
    dst = edge_index[1]

    wt_e = jnp.transpose(lin_edge_w, (0, 2, 1))
    be = lin_edge_b[:, None, :]
    e_list = _edge_embed(edge_attr, wt_e, be)

    zeros = jnp.zeros((N, D), jnp.float32)
    sc_edge = _make_sc_edge(N, D, E)

    h = x
    for l in range(L):
        agg = sc_edge(h, e_list[l], src, dst, zeros)
        h = _node_mlp(
            h, agg,
            mlp_w1[l].T, mlp_b1[l][None], mlp_bn_g[l][None], mlp_bn_b[l][None],
            mlp_w2[l].T, mlp_b2[l][None], norm_g[l][None], norm_b[l][None],
            l < L - 1,
        )
    return h
